# Initial kernel scaffold; baseline (speedup 1.0000x reference)
import jax
import jax.numpy as jnp
from jax.experimental import pallas as pl


def kernel(x_recipe, usr_emb, rcp_emb, edge_weight_usr_rcp, edge_weight_rcp_usr, edge_index_usr_rcp, edge_index_rcp_usr):
    raise NotImplementedError("write your pallas kernel here")


# speedup vs baseline: 2.9682x; 2.9682x over previous
import functools

import jax
import jax.numpy as jnp
from jax import lax
from jax.experimental import pallas as pl
from jax.experimental.pallas import tpu as pltpu
from jax.experimental.pallas import tpu_sc as plsc

N_NODE = 50000
DIM = 64
E = 800000
N_LAYERS = 3
ALPHA = 1.0 / (N_LAYERS + 1)

NC = 2
NS = 16
L = 16

HALF = N_NODE // 2
TPT = 1568
ACC_ROWS = NS * TPT
NPAD = NC * ACC_ROWS
PAD_OFF = ACC_ROWS - HALF

EPT = E // NS
G = 128
NGRP = EPT // G
REM = EPT - NGRP * G
RC = 8

_MESH = plsc.VectorSubcoreMesh(core_axis_name="c", subcore_axis_name="s")
_PARAMS = pltpu.CompilerParams(needs_layout_passes=False,
                               use_tc_tiling_on_sc=False)


def _deg_body(dst_hbm, invd_hbm, didx_v, didx_r, ones_v, chunk_v, out_v,
              acc_sh):
    c = lax.axis_index("c")
    s = lax.axis_index("s")
    half_base = c * HALF
    row0 = s * TPT
    zero16 = jnp.zeros((L,), jnp.float32)

    def _zr(j, _):
        ones_v[j, pl.ds(0, L)] = zero16
        return 0
    lax.fori_loop(0, G, _zr, 0)

    def _zc(j, _):
        pltpu.sync_copy(ones_v, acc_sh.at[pl.ds(row0 + j * G, G)])
        return 0
    lax.fori_loop(0, TPT // G, _zc, 0)
    pltpu.sync_copy(ones_v.at[pl.ds(0, TPT - (TPT // G) * G)],
                    acc_sh.at[pl.ds(row0 + (TPT // G) * G,
                                    TPT - (TPT // G) * G)])

    patt = jnp.where(lax.iota(jnp.int32, L) == 0, 1.0, 0.0).astype(jnp.float32)

    def _sp(j, _):
        ones_v[j, pl.ds(0, L)] = patt
        return 0
    lax.fori_loop(0, G, _sp, 0)
    plsc.subcore_barrier()

    ebase = s * EPT
    trash = HALF + s

    def _remap(ref, n16):
        for k in range(n16):
            sl = pl.ds(k * L, L)
            di = ref[sl]
            dl = di - half_base
            inb = (dl >= 0) & (dl < HALF)
            ref[sl] = jnp.where(inb, dl, trash)

    def _grp(g, _):
        b = ebase + g * G
        pltpu.sync_copy(dst_hbm.at[pl.ds(b, G)], didx_v)
        _remap(didx_v, G // L)
        pltpu.sync_copy(ones_v, acc_sh.at[didx_v], add=True)
        return 0
    lax.fori_loop(0, NGRP, _grp, 0)
    pltpu.sync_copy(dst_hbm.at[pl.ds(ebase + NGRP * G, REM)], didx_r)
    _remap(didx_r, REM // L)
    pltpu.sync_copy(ones_v.at[pl.ds(0, REM)], acc_sh.at[didx_r], add=True)
    plsc.subcore_barrier()

    lane0 = lax.iota(jnp.int32, L) == 0

    def _inv(j, _):
        r = row0 + j * L
        pltpu.sync_copy(acc_sh.at[pl.ds(r, L)], chunk_v)
        for i in range(L):
            row = chunk_v[i, pl.ds(0, L)]
            iv_row = 1.0 / jnp.maximum(row, 1.0)
            plsc.store_scatter(out_v, [jnp.full((L,), i, jnp.int32)],
                               iv_row, mask=lane0)
        pltpu.sync_copy(out_v, invd_hbm.at[pl.ds(c * ACC_ROWS + r, L)])
        return 0
    lax.fori_loop(0, TPT // L, _inv, 0)


_deg_call = pl.kernel(
    _deg_body,
    out_type=jax.ShapeDtypeStruct((NPAD,), jnp.float32),
    mesh=_MESH,
    compiler_params=_PARAMS,
    scratch_types=[
        pltpu.VMEM((G,), jnp.int32),
        pltpu.VMEM((REM,), jnp.int32),
        pltpu.VMEM((G, L), jnp.float32),
        pltpu.VMEM((L, L), jnp.float32),
        pltpu.VMEM((L,), jnp.float32),
        pltpu.VMEM_SHARED((ACC_ROWS, L), jnp.float32),
    ],
)


def _conv_body(x_hbm, src_hbm, dst_hbm, invd_hbm, y_hbm,
               sidx_v, didx_v, sidx_r, didx_r, rows_v, nrm_v, invd_v, sem,
               acc_sh):
    c = lax.axis_index("c")
    s = lax.axis_index("s")
    half_base = c * HALF
    row0 = s * TPT
    zero16 = jnp.zeros((L,), jnp.float32)

    def _zr(j, _):
        for k in range(DIM // L):
            rows_v[j, pl.ds(k * L, L)] = zero16
        return 0
    lax.fori_loop(0, G, _zr, 0)

    def _zc(j, _):
        pltpu.sync_copy(rows_v, acc_sh.at[pl.ds(row0 + j * G, G)])
        return 0
    lax.fori_loop(0, TPT // G, _zc, 0)
    pltpu.sync_copy(rows_v.at[pl.ds(0, TPT - (TPT // G) * G)],
                    acc_sh.at[pl.ds(row0 + (TPT // G) * G,
                                    TPT - (TPT // G) * G)])
    plsc.subcore_barrier()

    ebase = s * EPT
    trash = HALF + s

    def _remap(sref, dref, n16):
        for k in range(n16):
            sl = pl.ds(k * L, L)
            si = sref[sl]
            sref[sl] = jnp.where(si >= HALF, si + PAD_OFF, si)
            di = dref[sl]
            dl = di - half_base
            inb = (dl >= 0) & (dl < HALF)
            dref[sl] = jnp.where(inb, dl, trash)

    def _grp(g, _):
        b = ebase + g * G
        pltpu.sync_copy(src_hbm.at[pl.ds(b, G)], sidx_v)
        pltpu.sync_copy(dst_hbm.at[pl.ds(b, G)], didx_v)
        _remap(sidx_v, didx_v, G // L)
        pltpu.async_copy(x_hbm.at[sidx_v], rows_v, sem).wait()
        pltpu.sync_copy(rows_v, acc_sh.at[didx_v], add=True)
        return 0
    lax.fori_loop(0, NGRP, _grp, 0)
    b = ebase + NGRP * G
    pltpu.sync_copy(src_hbm.at[pl.ds(b, REM)], sidx_r)
    pltpu.sync_copy(dst_hbm.at[pl.ds(b, REM)], didx_r)
    _remap(sidx_r, didx_r, REM // L)
    pltpu.async_copy(x_hbm.at[sidx_r], rows_v.at[pl.ds(0, REM)], sem).wait()
    pltpu.sync_copy(rows_v.at[pl.ds(0, REM)], acc_sh.at[didx_r], add=True)
    plsc.subcore_barrier()

    def _nrm(j, _):
        r = row0 + j * L
        pltpu.sync_copy(acc_sh.at[pl.ds(r, L)], nrm_v)
        pltpu.sync_copy(invd_hbm.at[pl.ds(c * ACC_ROWS + r, L)], invd_v)
        ivv = invd_v[pl.ds(0, L)]
        for i in range(L):
            iv = jnp.full((L,), ivv[i], jnp.float32)
            for k in range(DIM // L):
                sl = pl.ds(k * L, L)
                nrm_v[i, sl] = nrm_v[i, sl] * iv
        pltpu.sync_copy(nrm_v, y_hbm.at[pl.ds(c * ACC_ROWS + r, L)])
        return 0
    lax.fori_loop(0, TPT // L, _nrm, 0)


_conv_call = pl.kernel(
    _conv_body,
    out_type=jax.ShapeDtypeStruct((NPAD, DIM), jnp.float32),
    mesh=_MESH,
    compiler_params=_PARAMS,
    scratch_types=[
        pltpu.VMEM((G,), jnp.int32),
        pltpu.VMEM((G,), jnp.int32),
        pltpu.VMEM((REM,), jnp.int32),
        pltpu.VMEM((REM,), jnp.int32),
        pltpu.VMEM((G, DIM), jnp.float32),
        pltpu.VMEM((L, DIM), jnp.float32),
        pltpu.VMEM((L,), jnp.float32),
        pltpu.SemaphoreType.DMA,
        pltpu.VMEM_SHARED((ACC_ROWS, DIM), jnp.float32),
    ],
)


def _final_body(x0_hbm, y1_hbm, y2_hbm, y3_hbm, out_hbm,
                a_v, b_v, c_v, d_v):
    c = lax.axis_index("c")
    s = lax.axis_index("s")
    w = c * NS + s
    r0 = w * TPT
    nch = jnp.minimum(TPT, N_NODE - r0) // RC

    def _chunk(j, _):
        rlog = r0 + j * RC
        rphy = rlog + jnp.where(rlog >= HALF, PAD_OFF, 0)
        pltpu.sync_copy(x0_hbm.at[pl.ds(rphy, RC)], a_v)
        pltpu.sync_copy(y1_hbm.at[pl.ds(rphy, RC)], b_v)
        pltpu.sync_copy(y2_hbm.at[pl.ds(rphy, RC)], c_v)
        pltpu.sync_copy(y3_hbm.at[pl.ds(rphy, RC)], d_v)
        for i in range(RC):
            for k in range(DIM // L):
                sl = pl.ds(k * L, L)
                a_v[i, sl] = (a_v[i, sl] + b_v[i, sl]
                              + c_v[i, sl] + d_v[i, sl]) * ALPHA
        pltpu.sync_copy(a_v, out_hbm.at[pl.ds(rlog, RC)])
        return 0
    lax.fori_loop(0, nch, _chunk, 0)


_final_call = pl.kernel(
    _final_body,
    out_type=jax.ShapeDtypeStruct((N_NODE, DIM), jnp.float32),
    mesh=_MESH,
    compiler_params=_PARAMS,
    scratch_types=[
        pltpu.VMEM((RC, DIM), jnp.float32),
        pltpu.VMEM((RC, DIM), jnp.float32),
        pltpu.VMEM((RC, DIM), jnp.float32),
        pltpu.VMEM((RC, DIM), jnp.float32),
    ],
)


def _pad(t):
    z = jnp.zeros((PAD_OFF, DIM), jnp.float32)
    return jnp.concatenate([t[:HALF], z, t[HALF:], z], axis=0)


def kernel(x_recipe, usr_emb, rcp_emb, edge_weight_usr_rcp,
           edge_weight_rcp_usr, edge_index_usr_rcp, edge_index_rcp_usr):
    del edge_weight_usr_rcp, edge_weight_rcp_usr
    src_ur = edge_index_usr_rcp[0]
    dst_ur = edge_index_usr_rcp[1]
    src_ru = edge_index_rcp_usr[0]
    dst_ru = edge_index_rcp_usr[1]

    usr0 = _pad(usr_emb)
    rcp0 = _pad(jnp.concatenate([rcp_emb, x_recipe], axis=1))

    invd_rcp = _deg_call(dst_ur)
    invd_usr = _deg_call(dst_ru)

    x_u = usr0
    ys_r = []
    ys_u = []
    for _ in range(N_LAYERS):
        x_r = _conv_call(x_u, src_ur, dst_ur, invd_rcp)
        ys_r.append(x_r)
        x_u = _conv_call(x_r, src_ru, dst_ru, invd_usr)
        ys_u.append(x_u)

    usr_out = _final_call(usr0, ys_u[0], ys_u[1], ys_u[2])
    rec_out = _final_call(rcp0, ys_r[0], ys_r[1], ys_r[2])
    return (usr_out, rec_out)


# speedup vs baseline: 5.1356x; 1.7302x over previous
import jax
import jax.numpy as jnp
from jax import lax
from jax.experimental import pallas as pl
from jax.experimental.pallas import tpu as pltpu
from jax.experimental.pallas import tpu_sc as plsc

N_NODE = 50000
DIM = 64
DH = 32
E = 800000
N_LAYERS = 3
ALPHA = 1.0 / (N_LAYERS + 1)

NC = 2
NS = 16
L = 16

HALF = N_NODE // 2
TPT = 1568
ACC_ROWS = NS * TPT
NPAD = NC * ACC_ROWS
PAD_OFF = ACC_ROWS - HALF

G = 128
K = 4
EPAD = 512 * TPT
GPT = EPAD // NS // G
R = GPT // K
EROWS = EPAD // G

TPT_F = 1600
RC_F = 40

_MESH = plsc.VectorSubcoreMesh(core_axis_name="c", subcore_axis_name="s")
_PARAMS = pltpu.CompilerParams(needs_layout_passes=False,
                               use_tc_tiling_on_sc=False)


def _remap_dst(dref, j, half_base, trash):
    for k in range(G // L):
        sl = pl.ds(k * L, L)
        di = dref[j, sl]
        dl = di - half_base
        inb = (dl >= 0) & (dl < HALF)
        dref[j, sl] = jnp.where(inb, dl, trash)


def _remap_src(sref, j):
    for k in range(G // L):
        sl = pl.ds(k * L, L)
        si = sref[j, sl]
        sref[j, sl] = jnp.where(si >= HALF, si + PAD_OFF, si)


def _deg_body(dst_hbm, invd_hbm, didx_a, didx_b, ones_v, chunk_v, out_v,
              sem_a, sem_b, acc_sh):
    c = lax.axis_index("c")
    s = lax.axis_index("s")
    half_base = c * HALF
    row0 = s * TPT
    trash = HALF + s
    gbase0 = s * GPT
    zero16 = jnp.zeros((L,), jnp.float32)

    def _zr(j, _):
        ones_v[j, pl.ds(0, L)] = zero16
        return 0
    lax.fori_loop(0, G, _zr, 0)

    def _zc(j, _):
        pltpu.sync_copy(ones_v, acc_sh.at[pl.ds(row0 + j * G, G)])
        return 0
    lax.fori_loop(0, TPT // G, _zc, 0)
    pltpu.sync_copy(ones_v.at[pl.ds(0, TPT - (TPT // G) * G)],
                    acc_sh.at[pl.ds(row0 + (TPT // G) * G,
                                    TPT - (TPT // G) * G)])

    patt = jnp.where(lax.iota(jnp.int32, L) == 0, 1.0, 0.0).astype(jnp.float32)

    def _sp(j, _):
        ones_v[j, pl.ds(0, L)] = patt
        return 0
    lax.fori_loop(0, G, _sp, 0)
    plsc.subcore_barrier()

    def _fire(didx, sem, r):
        gb = gbase0 + r * K
        pltpu.sync_copy(dst_hbm.at[pl.ds(gb, K)], didx)
        for j in range(K):
            _remap_dst(didx, j, half_base, trash)
            pltpu.async_copy(ones_v, acc_sh.at[didx.at[j]], sem, add=True)

    def _drain(didx, sem):
        for j in range(K):
            pltpu.make_async_copy(ones_v, acc_sh.at[didx.at[j]], sem).wait()

    _fire(didx_a, sem_a, 0)
    _fire(didx_b, sem_b, 1)

    def _pair(i, _):
        r = 2 + 2 * i
        _drain(didx_a, sem_a)
        _fire(didx_a, sem_a, r)
        _drain(didx_b, sem_b)
        _fire(didx_b, sem_b, r + 1)
        return 0
    lax.fori_loop(0, (R - 2) // 2, _pair, 0)
    _drain(didx_a, sem_a)
    _drain(didx_b, sem_b)
    plsc.subcore_barrier()

    lane0 = lax.iota(jnp.int32, L) == 0

    def _inv(j, _):
        r = row0 + j * L
        pltpu.sync_copy(acc_sh.at[pl.ds(r, L)], chunk_v)
        for i in range(L):
            row = chunk_v[i, pl.ds(0, L)]
            iv_row = 1.0 / jnp.maximum(row, 1.0)
            plsc.store_scatter(out_v, [jnp.full((L,), i, jnp.int32)],
                               iv_row, mask=lane0)
        pltpu.sync_copy(out_v, invd_hbm.at[pl.ds(c * ACC_ROWS + r, L)])
        return 0
    lax.fori_loop(0, TPT // L, _inv, 0)


_deg_call = pl.kernel(
    _deg_body,
    out_type=jax.ShapeDtypeStruct((NPAD,), jnp.float32),
    mesh=_MESH,
    compiler_params=_PARAMS,
    scratch_types=[
        pltpu.VMEM((K, G), jnp.int32),
        pltpu.VMEM((K, G), jnp.int32),
        pltpu.VMEM((G, L), jnp.float32),
        pltpu.VMEM((L, L), jnp.float32),
        pltpu.VMEM((L,), jnp.float32),
        pltpu.SemaphoreType.DMA,
        pltpu.SemaphoreType.DMA,
        pltpu.VMEM_SHARED((ACC_ROWS, L), jnp.float32),
    ],
)


def _conv_body(xlo_hbm, xhi_hbm, src_hbm, dst_hbm, invd_hbm,
               ylo_hbm, yhi_hbm,
               sidx_a, didx_a, rows_a, sidx_b, didx_b, rows_b,
               nrm_v, invd_v, gsem_a, gsem_b, acc_sh):
    c = lax.axis_index("c")
    s = lax.axis_index("s")
    half_base = c * HALF
    row0 = s * TPT
    trash = HALF + s
    gbase0 = s * GPT
    zero16 = jnp.zeros((L,), jnp.float32)

    for x_hbm, y_hbm in ((xlo_hbm, ylo_hbm), (xhi_hbm, yhi_hbm)):
        def _zr(j, _):
            for k in range(DH // L):
                rows_a[0, j, pl.ds(k * L, L)] = zero16
            return 0
        lax.fori_loop(0, G, _zr, 0)
        zrows = rows_a.at[0]

        def _zc(j, _):
            pltpu.sync_copy(zrows, acc_sh.at[pl.ds(row0 + j * G, G)])
            return 0
        lax.fori_loop(0, TPT // G, _zc, 0)
        pltpu.sync_copy(zrows.at[pl.ds(0, TPT - (TPT // G) * G)],
                        acc_sh.at[pl.ds(row0 + (TPT // G) * G,
                                        TPT - (TPT // G) * G)])
        plsc.subcore_barrier()

        def _fire_g(sidx, didx, rows, gsem, r):
            gb = gbase0 + r * K
            pltpu.sync_copy(src_hbm.at[pl.ds(gb, K)], sidx)
            pltpu.sync_copy(dst_hbm.at[pl.ds(gb, K)], didx)
            for j in range(K):
                _remap_src(sidx, j)
                _remap_dst(didx, j, half_base, trash)
                pltpu.async_copy(x_hbm.at[sidx.at[j]], rows.at[j], gsem)

        def _drain_g(sidx, rows, gsem):
            for j in range(K):
                pltpu.make_async_copy(x_hbm.at[sidx.at[j]], rows.at[j],
                                      gsem).wait()

        def _scat(didx, rows):
            def _one(j, _):
                pltpu.sync_copy(rows.at[j], acc_sh.at[didx.at[j]], add=True)
                return 0
            lax.fori_loop(0, K, _one, 0)

        _fire_g(sidx_a, didx_a, rows_a, gsem_a, 0)

        def _pair(i, _):
            r = 1 + 2 * i
            _fire_g(sidx_b, didx_b, rows_b, gsem_b, r)
            _drain_g(sidx_a, rows_a, gsem_a)
            _scat(didx_a, rows_a)
            _fire_g(sidx_a, didx_a, rows_a, gsem_a, r + 1)
            _drain_g(sidx_b, rows_b, gsem_b)
            _scat(didx_b, rows_b)
            return 0
        lax.fori_loop(0, (R - 2) // 2, _pair, 0)
        _fire_g(sidx_b, didx_b, rows_b, gsem_b, R - 1)
        _drain_g(sidx_a, rows_a, gsem_a)
        _scat(didx_a, rows_a)
        _drain_g(sidx_b, rows_b, gsem_b)
        _scat(didx_b, rows_b)
        plsc.subcore_barrier()

        def _nrm(j, _):
            r = row0 + j * L
            pltpu.sync_copy(acc_sh.at[pl.ds(r, L)], nrm_v)
            pltpu.sync_copy(invd_hbm.at[pl.ds(c * ACC_ROWS + r, L)], invd_v)
            ivv = invd_v[pl.ds(0, L)]
            for i in range(L):
                iv = jnp.full((L,), ivv[i], jnp.float32)
                for k in range(DH // L):
                    sl = pl.ds(k * L, L)
                    nrm_v[i, sl] = nrm_v[i, sl] * iv
            pltpu.sync_copy(nrm_v, y_hbm.at[pl.ds(c * ACC_ROWS + r, L)])
            return 0
        lax.fori_loop(0, TPT // L, _nrm, 0)
        plsc.subcore_barrier()


_conv_call = pl.kernel(
    _conv_body,
    out_type=(jax.ShapeDtypeStruct((NPAD, DH), jnp.float32),
              jax.ShapeDtypeStruct((NPAD, DH), jnp.float32)),
    mesh=_MESH,
    compiler_params=_PARAMS,
    scratch_types=[
        pltpu.VMEM((K, G), jnp.int32),
        pltpu.VMEM((K, G), jnp.int32),
        pltpu.VMEM((K, G, DH), jnp.float32),
        pltpu.VMEM((K, G), jnp.int32),
        pltpu.VMEM((K, G), jnp.int32),
        pltpu.VMEM((K, G, DH), jnp.float32),
        pltpu.VMEM((L, DH), jnp.float32),
        pltpu.VMEM((L,), jnp.float32),
        pltpu.SemaphoreType.DMA,
        pltpu.SemaphoreType.DMA,
        pltpu.VMEM_SHARED((ACC_ROWS, DH), jnp.float32),
    ],
)


def _final_body(x0l, x0h, y1l, y1h, y2l, y2h, y3l, y3h, out_hbm,
                al_v, bl_v, cl_v, dl_v, ah_v, bh_v, ch_v, dh_v, out_v, sem):
    c = lax.axis_index("c")
    s = lax.axis_index("s")
    w = c * NS + s
    r0 = w * TPT_F
    nch = jnp.maximum(jnp.minimum(TPT_F, N_NODE - r0), 0) // RC_F

    def _chunk(j, _):
        rlog = r0 + j * RC_F
        rphy = rlog + jnp.where(rlog >= HALF, PAD_OFF, 0)
        srcs = ((x0l, al_v), (y1l, bl_v), (y2l, cl_v), (y3l, dl_v),
                (x0h, ah_v), (y1h, bh_v), (y2h, ch_v), (y3h, dh_v))
        for t, buf in srcs:
            pltpu.async_copy(t.at[pl.ds(rphy, RC_F)], buf, sem)
        for t, buf in srcs:
            pltpu.make_async_copy(t.at[pl.ds(rphy, RC_F)], buf, sem).wait()
        for i in range(RC_F):
            for k in range(DH // L):
                sl = pl.ds(k * L, L)
                out_v[i, pl.ds(k * L, L)] = (
                    al_v[i, sl] + bl_v[i, sl] + cl_v[i, sl] + dl_v[i, sl]
                ) * ALPHA
                out_v[i, pl.ds(DH + k * L, L)] = (
                    ah_v[i, sl] + bh_v[i, sl] + ch_v[i, sl] + dh_v[i, sl]
                ) * ALPHA
        pltpu.sync_copy(out_v, out_hbm.at[pl.ds(rlog, RC_F)])
        return 0
    lax.fori_loop(0, nch, _chunk, 0)


_final_call = pl.kernel(
    _final_body,
    out_type=jax.ShapeDtypeStruct((N_NODE, DIM), jnp.float32),
    mesh=_MESH,
    compiler_params=_PARAMS,
    scratch_types=[
        pltpu.VMEM((RC_F, DH), jnp.float32),
        pltpu.VMEM((RC_F, DH), jnp.float32),
        pltpu.VMEM((RC_F, DH), jnp.float32),
        pltpu.VMEM((RC_F, DH), jnp.float32),
        pltpu.VMEM((RC_F, DH), jnp.float32),
        pltpu.VMEM((RC_F, DH), jnp.float32),
        pltpu.VMEM((RC_F, DH), jnp.float32),
        pltpu.VMEM((RC_F, DH), jnp.float32),
        pltpu.VMEM((RC_F, DIM), jnp.float32),
        pltpu.SemaphoreType.DMA,
    ],
)


def _pad(t):
    z = jnp.zeros((PAD_OFF, DH), jnp.float32)
    return jnp.concatenate([t[:HALF], z, t[HALF:], z], axis=0)


def _pad_edges(ei):
    npad = EPAD - E
    src = jnp.concatenate([ei[0], jnp.zeros((npad,), jnp.int32)])
    dst = jnp.concatenate([ei[1], jnp.full((npad,), -1, jnp.int32)])
    return src.reshape(EROWS, G), dst.reshape(EROWS, G)


def kernel(x_recipe, usr_emb, rcp_emb, edge_weight_usr_rcp,
           edge_weight_rcp_usr, edge_index_usr_rcp, edge_index_rcp_usr):
    del edge_weight_usr_rcp, edge_weight_rcp_usr
    src_ur, dst_ur = _pad_edges(edge_index_usr_rcp)
    src_ru, dst_ru = _pad_edges(edge_index_rcp_usr)

    usr0 = (_pad(usr_emb[:, :DH]), _pad(usr_emb[:, DH:]))
    rcp_x = jnp.concatenate([rcp_emb, x_recipe], axis=1)
    rcp0 = (_pad(rcp_x[:, :DH]), _pad(rcp_x[:, DH:]))

    invd_rcp = _deg_call(dst_ur)
    invd_usr = _deg_call(dst_ru)

    x_u = usr0
    ys_r = []
    ys_u = []
    for _ in range(N_LAYERS):
        x_r = _conv_call(x_u[0], x_u[1], src_ur, dst_ur, invd_rcp)
        ys_r.append(x_r)
        x_u = _conv_call(x_r[0], x_r[1], src_ru, dst_ru, invd_usr)
        ys_u.append(x_u)

    usr_out = _final_call(usr0[0], usr0[1], ys_u[0][0], ys_u[0][1],
                          ys_u[1][0], ys_u[1][1], ys_u[2][0], ys_u[2][1])
    rec_out = _final_call(rcp0[0], rcp0[1], ys_r[0][0], ys_r[0][1],
                          ys_r[1][0], ys_r[1][1], ys_r[2][0], ys_r[2][1])
    return (usr_out, rec_out)


# speedup vs baseline: 5.8553x; 1.1402x over previous
import jax
import jax.numpy as jnp
from jax import lax
from jax.experimental import pallas as pl
from jax.experimental.pallas import tpu as pltpu
from jax.experimental.pallas import tpu_sc as plsc

N_NODE = 50000
DIM = 64
DH = 32
E = 800000
N_LAYERS = 3
ALPHA = 1.0 / (N_LAYERS + 1)

NC = 2
NS = 16
L = 16

HALF = N_NODE // 2
TPT = 1568
ACC_ROWS = NS * TPT
NPAD = NC * ACC_ROWS
PAD_OFF = ACC_ROWS - HALF

G = 128
K = 8
EPAD = 512 * TPT
GPT = EPAD // NS // G
R = GPT // K
EROWS = EPAD // G

TPT_F = 1600
RC_F = 40

_MESH = plsc.VectorSubcoreMesh(core_axis_name="c", subcore_axis_name="s")
_PARAMS = pltpu.CompilerParams(needs_layout_passes=False,
                               use_tc_tiling_on_sc=False)


def _remap_dst(dref, j, half_base, trash):
    for k in range(G // L):
        sl = pl.ds(k * L, L)
        di = dref[j, sl]
        dl = di - half_base
        inb = (dl >= 0) & (dl < HALF)
        dref[j, sl] = jnp.where(inb, dl, trash)


def _remap_src(sref, j):
    for k in range(G // L):
        sl = pl.ds(k * L, L)
        si = sref[j, sl]
        sref[j, sl] = jnp.where(si >= HALF, si + PAD_OFF, si)


def _deg_body(dst_hbm, invd_hbm, didx_a, didx_b, ones_v, chunk_v, out_v,
              sem_a, sem_b, acc_sh):
    c = lax.axis_index("c")
    s = lax.axis_index("s")
    half_base = c * HALF
    row0 = s * TPT
    trash = HALF + s
    gbase0 = s * GPT
    zero16 = jnp.zeros((L,), jnp.float32)

    def _zr(j, _):
        ones_v[j, pl.ds(0, L)] = zero16
        return 0
    lax.fori_loop(0, G, _zr, 0)

    def _zc(j, _):
        pltpu.sync_copy(ones_v, acc_sh.at[pl.ds(row0 + j * G, G)])
        return 0
    lax.fori_loop(0, TPT // G, _zc, 0)
    pltpu.sync_copy(ones_v.at[pl.ds(0, TPT - (TPT // G) * G)],
                    acc_sh.at[pl.ds(row0 + (TPT // G) * G,
                                    TPT - (TPT // G) * G)])

    patt = jnp.where(lax.iota(jnp.int32, L) == 0, 1.0, 0.0).astype(jnp.float32)

    def _sp(j, _):
        ones_v[j, pl.ds(0, L)] = patt
        return 0
    lax.fori_loop(0, G, _sp, 0)
    plsc.subcore_barrier()

    def _fire(didx, sem, r):
        gb = gbase0 + r * K
        pltpu.sync_copy(dst_hbm.at[pl.ds(gb, K)], didx)
        for j in range(K):
            _remap_dst(didx, j, half_base, trash)
            pltpu.async_copy(ones_v, acc_sh.at[didx.at[j]], sem, add=True)

    def _drain(didx, sem):
        for j in range(K):
            pltpu.make_async_copy(ones_v, acc_sh.at[didx.at[j]], sem).wait()

    _fire(didx_a, sem_a, 0)
    _fire(didx_b, sem_b, 1)

    def _pair(i, _):
        r = 2 + 2 * i
        _drain(didx_a, sem_a)
        _fire(didx_a, sem_a, r)
        _drain(didx_b, sem_b)
        _fire(didx_b, sem_b, r + 1)
        return 0
    lax.fori_loop(0, (R - 2) // 2, _pair, 0)
    _drain(didx_a, sem_a)
    _fire(didx_a, sem_a, R - 1)
    _drain(didx_b, sem_b)
    _drain(didx_a, sem_a)
    plsc.subcore_barrier()

    lane0 = lax.iota(jnp.int32, L) == 0

    def _inv(j, _):
        r = row0 + j * L
        pltpu.sync_copy(acc_sh.at[pl.ds(r, L)], chunk_v)
        for i in range(L):
            row = chunk_v[i, pl.ds(0, L)]
            iv_row = 1.0 / jnp.maximum(row, 1.0)
            plsc.store_scatter(out_v, [jnp.full((L,), i, jnp.int32)],
                               iv_row, mask=lane0)
        pltpu.sync_copy(out_v, invd_hbm.at[pl.ds(c * ACC_ROWS + r, L)])
        return 0
    lax.fori_loop(0, TPT // L, _inv, 0)


_deg_call = pl.kernel(
    _deg_body,
    out_type=jax.ShapeDtypeStruct((NPAD,), jnp.float32),
    mesh=_MESH,
    compiler_params=_PARAMS,
    scratch_types=[
        pltpu.VMEM((K, G), jnp.int32),
        pltpu.VMEM((K, G), jnp.int32),
        pltpu.VMEM((G, L), jnp.float32),
        pltpu.VMEM((L, L), jnp.float32),
        pltpu.VMEM((L,), jnp.float32),
        pltpu.SemaphoreType.DMA,
        pltpu.SemaphoreType.DMA,
        pltpu.VMEM_SHARED((ACC_ROWS, L), jnp.float32),
    ],
)


def _conv_body(xlo_hbm, xhi_hbm, src_hbm, dst_hbm, invd_hbm,
               ylo_hbm, yhi_hbm,
               sidx_a, didx_a, rows_a, sidx_b, didx_b, rows_b,
               nrm_v, invd_v, gsem_a, gsem_b, acc_sh):
    c = lax.axis_index("c")
    s = lax.axis_index("s")
    half_base = c * HALF
    row0 = s * TPT
    trash = HALF + s
    gbase0 = s * GPT
    zero16 = jnp.zeros((L,), jnp.float32)

    for x_hbm, y_hbm in ((xlo_hbm, ylo_hbm), (xhi_hbm, yhi_hbm)):
        def _zr(j, _):
            for k in range(DH // L):
                rows_a[0, j, pl.ds(k * L, L)] = zero16
            return 0
        lax.fori_loop(0, G, _zr, 0)
        zrows = rows_a.at[0]

        def _zc(j, _):
            pltpu.sync_copy(zrows, acc_sh.at[pl.ds(row0 + j * G, G)])
            return 0
        lax.fori_loop(0, TPT // G, _zc, 0)
        pltpu.sync_copy(zrows.at[pl.ds(0, TPT - (TPT // G) * G)],
                        acc_sh.at[pl.ds(row0 + (TPT // G) * G,
                                        TPT - (TPT // G) * G)])
        plsc.subcore_barrier()

        def _fire_g(sidx, didx, rows, gsem, r):
            gb = gbase0 + r * K
            pltpu.sync_copy(src_hbm.at[pl.ds(gb, K)], sidx)
            pltpu.sync_copy(dst_hbm.at[pl.ds(gb, K)], didx)
            for j in range(K):
                _remap_src(sidx, j)
                _remap_dst(didx, j, half_base, trash)
                pltpu.async_copy(x_hbm.at[sidx.at[j]], rows.at[j], gsem)

        def _drain_g(sidx, rows, gsem):
            for j in range(K):
                pltpu.make_async_copy(x_hbm.at[sidx.at[j]], rows.at[j],
                                      gsem).wait()

        def _scat(didx, rows):
            def _one(j, _):
                pltpu.sync_copy(rows.at[j], acc_sh.at[didx.at[j]], add=True)
                return 0
            lax.fori_loop(0, K, _one, 0)

        _fire_g(sidx_a, didx_a, rows_a, gsem_a, 0)

        def _pair(i, _):
            r = 1 + 2 * i
            _fire_g(sidx_b, didx_b, rows_b, gsem_b, r)
            _drain_g(sidx_a, rows_a, gsem_a)
            _scat(didx_a, rows_a)
            _fire_g(sidx_a, didx_a, rows_a, gsem_a, r + 1)
            _drain_g(sidx_b, rows_b, gsem_b)
            _scat(didx_b, rows_b)
            return 0
        lax.fori_loop(0, (R - 1) // 2, _pair, 0)
        _drain_g(sidx_a, rows_a, gsem_a)
        _scat(didx_a, rows_a)
        plsc.subcore_barrier()

        def _nrm(j, _):
            r = row0 + j * L
            pltpu.sync_copy(acc_sh.at[pl.ds(r, L)], nrm_v)
            pltpu.sync_copy(invd_hbm.at[pl.ds(c * ACC_ROWS + r, L)], invd_v)
            ivv = invd_v[pl.ds(0, L)]
            for i in range(L):
                iv = jnp.full((L,), ivv[i], jnp.float32)
                for k in range(DH // L):
                    sl = pl.ds(k * L, L)
                    nrm_v[i, sl] = nrm_v[i, sl] * iv
            pltpu.sync_copy(nrm_v, y_hbm.at[pl.ds(c * ACC_ROWS + r, L)])
            return 0
        lax.fori_loop(0, TPT // L, _nrm, 0)
        plsc.subcore_barrier()


_conv_call = pl.kernel(
    _conv_body,
    out_type=(jax.ShapeDtypeStruct((NPAD, DH), jnp.float32),
              jax.ShapeDtypeStruct((NPAD, DH), jnp.float32)),
    mesh=_MESH,
    compiler_params=_PARAMS,
    scratch_types=[
        pltpu.VMEM((K, G), jnp.int32),
        pltpu.VMEM((K, G), jnp.int32),
        pltpu.VMEM((K, G, DH), jnp.float32),
        pltpu.VMEM((K, G), jnp.int32),
        pltpu.VMEM((K, G), jnp.int32),
        pltpu.VMEM((K, G, DH), jnp.float32),
        pltpu.VMEM((L, DH), jnp.float32),
        pltpu.VMEM((L,), jnp.float32),
        pltpu.SemaphoreType.DMA,
        pltpu.SemaphoreType.DMA,
        pltpu.VMEM_SHARED((ACC_ROWS, DH), jnp.float32),
    ],
)


def _final_body(x0l, x0h, y1l, y1h, y2l, y2h, y3l, y3h, out_hbm,
                al_v, bl_v, cl_v, dl_v, ah_v, bh_v, ch_v, dh_v, out_v, sem):
    c = lax.axis_index("c")
    s = lax.axis_index("s")
    w = c * NS + s
    r0 = w * TPT_F
    nch = jnp.maximum(jnp.minimum(TPT_F, N_NODE - r0), 0) // RC_F

    def _chunk(j, _):
        rlog = r0 + j * RC_F
        rphy = rlog + jnp.where(rlog >= HALF, PAD_OFF, 0)
        srcs = ((x0l, al_v), (y1l, bl_v), (y2l, cl_v), (y3l, dl_v),
                (x0h, ah_v), (y1h, bh_v), (y2h, ch_v), (y3h, dh_v))
        for t, buf in srcs:
            pltpu.async_copy(t.at[pl.ds(rphy, RC_F)], buf, sem)
        for t, buf in srcs:
            pltpu.make_async_copy(t.at[pl.ds(rphy, RC_F)], buf, sem).wait()
        for i in range(RC_F):
            for k in range(DH // L):
                sl = pl.ds(k * L, L)
                out_v[i, pl.ds(k * L, L)] = (
                    al_v[i, sl] + bl_v[i, sl] + cl_v[i, sl] + dl_v[i, sl]
                ) * ALPHA
                out_v[i, pl.ds(DH + k * L, L)] = (
                    ah_v[i, sl] + bh_v[i, sl] + ch_v[i, sl] + dh_v[i, sl]
                ) * ALPHA
        pltpu.sync_copy(out_v, out_hbm.at[pl.ds(rlog, RC_F)])
        return 0
    lax.fori_loop(0, nch, _chunk, 0)


_final_call = pl.kernel(
    _final_body,
    out_type=jax.ShapeDtypeStruct((N_NODE, DIM), jnp.float32),
    mesh=_MESH,
    compiler_params=_PARAMS,
    scratch_types=[
        pltpu.VMEM((RC_F, DH), jnp.float32),
        pltpu.VMEM((RC_F, DH), jnp.float32),
        pltpu.VMEM((RC_F, DH), jnp.float32),
        pltpu.VMEM((RC_F, DH), jnp.float32),
        pltpu.VMEM((RC_F, DH), jnp.float32),
        pltpu.VMEM((RC_F, DH), jnp.float32),
        pltpu.VMEM((RC_F, DH), jnp.float32),
        pltpu.VMEM((RC_F, DH), jnp.float32),
        pltpu.VMEM((RC_F, DIM), jnp.float32),
        pltpu.SemaphoreType.DMA,
    ],
)


def _pad(t):
    z = jnp.zeros((PAD_OFF, DH), jnp.float32)
    return jnp.concatenate([t[:HALF], z, t[HALF:], z], axis=0)


def _pad_edges(ei):
    npad = EPAD - E
    src = jnp.concatenate([ei[0], jnp.zeros((npad,), jnp.int32)])
    dst = jnp.concatenate([ei[1], jnp.full((npad,), -1, jnp.int32)])
    return src.reshape(EROWS, G), dst.reshape(EROWS, G)


def kernel(x_recipe, usr_emb, rcp_emb, edge_weight_usr_rcp,
           edge_weight_rcp_usr, edge_index_usr_rcp, edge_index_rcp_usr):
    del edge_weight_usr_rcp, edge_weight_rcp_usr
    src_ur, dst_ur = _pad_edges(edge_index_usr_rcp)
    src_ru, dst_ru = _pad_edges(edge_index_rcp_usr)

    usr0 = (_pad(usr_emb[:, :DH]), _pad(usr_emb[:, DH:]))
    rcp_x = jnp.concatenate([rcp_emb, x_recipe], axis=1)
    rcp0 = (_pad(rcp_x[:, :DH]), _pad(rcp_x[:, DH:]))

    invd_rcp = _deg_call(dst_ur)
    invd_usr = _deg_call(dst_ru)

    x_u = usr0
    ys_r = []
    ys_u = []
    for _ in range(N_LAYERS):
        x_r = _conv_call(x_u[0], x_u[1], src_ur, dst_ur, invd_rcp)
        ys_r.append(x_r)
        x_u = _conv_call(x_r[0], x_r[1], src_ru, dst_ru, invd_usr)
        ys_u.append(x_u)

    usr_out = _final_call(usr0[0], usr0[1], ys_u[0][0], ys_u[0][1],
                          ys_u[1][0], ys_u[1][1], ys_u[2][0], ys_u[2][1])
    rec_out = _final_call(rcp0[0], rcp0[1], ys_r[0][0], ys_r[0][1],
                          ys_r[1][0], ys_r[1][1], ys_r[2][0], ys_r[2][1])
    return (usr_out, rec_out)


# speedup vs baseline: 5.9811x; 1.0215x over previous
import jax
import jax.numpy as jnp
from jax import lax
from jax.experimental import pallas as pl
from jax.experimental.pallas import tpu as pltpu
from jax.experimental.pallas import tpu_sc as plsc

N_NODE = 50000
DIM = 64
DH = 32
E = 800000
N_LAYERS = 3
ALPHA = 1.0 / (N_LAYERS + 1)

NC = 2
NS = 16
L = 16

HALF = N_NODE // 2
TPT = 1568
ACC_ROWS = NS * TPT
NPAD = NC * ACC_ROWS
PAD_OFF = ACC_ROWS - HALF

G = 128
K = 8
EPAD = 512 * TPT
GPT = EPAD // NS // G
R = GPT // K
EROWS = EPAD // G

TPT_F = 1600
RC_F = 40

_MESH = plsc.VectorSubcoreMesh(core_axis_name="c", subcore_axis_name="s")
_PARAMS = pltpu.CompilerParams(needs_layout_passes=False,
                               use_tc_tiling_on_sc=False)


def _remap_dst(dref, j, half_base, trash):
    for k in range(G // L):
        sl = pl.ds(k * L, L)
        di = dref[j, sl]
        dl = di - half_base
        inb = (dl >= 0) & (dl < HALF)
        dref[j, sl] = jnp.where(inb, dl, trash)


def _remap_src(sref, j):
    for k in range(G // L):
        sl = pl.ds(k * L, L)
        si = sref[j, sl]
        sref[j, sl] = jnp.where(si >= HALF, si + PAD_OFF, si)


def _deg_body(dst_hbm, invd_hbm, didx_a, didx_b, ones_v, chunk_v, out_v,
              sem_a, sem_b, acc_sh):
    c = lax.axis_index("c")
    s = lax.axis_index("s")
    half_base = c * HALF
    row0 = s * TPT
    trash = HALF + s
    gbase0 = s * GPT
    zero16 = jnp.zeros((L,), jnp.float32)

    def _zr(j, _):
        ones_v[j, pl.ds(0, L)] = zero16
        return 0
    lax.fori_loop(0, G, _zr, 0)

    def _zc(j, _):
        pltpu.sync_copy(ones_v, acc_sh.at[pl.ds(row0 + j * G, G)])
        return 0
    lax.fori_loop(0, TPT // G, _zc, 0)
    pltpu.sync_copy(ones_v.at[pl.ds(0, TPT - (TPT // G) * G)],
                    acc_sh.at[pl.ds(row0 + (TPT // G) * G,
                                    TPT - (TPT // G) * G)])

    patt = jnp.where(lax.iota(jnp.int32, L) == 0, 1.0, 0.0).astype(jnp.float32)

    def _sp(j, _):
        ones_v[j, pl.ds(0, L)] = patt
        return 0
    lax.fori_loop(0, G, _sp, 0)
    plsc.subcore_barrier()

    def _fire(didx, sem, r):
        gb = gbase0 + r * K
        pltpu.sync_copy(dst_hbm.at[pl.ds(gb, K)], didx)
        for j in range(K):
            _remap_dst(didx, j, half_base, trash)
            pltpu.async_copy(ones_v, acc_sh.at[didx.at[j]], sem, add=True)

    def _drain(didx, sem):
        for j in range(K):
            pltpu.make_async_copy(ones_v, acc_sh.at[didx.at[j]], sem).wait()

    _fire(didx_a, sem_a, 0)
    _fire(didx_b, sem_b, 1)

    def _pair(i, _):
        r = 2 + 2 * i
        _drain(didx_a, sem_a)
        _fire(didx_a, sem_a, r)
        _drain(didx_b, sem_b)
        _fire(didx_b, sem_b, r + 1)
        return 0
    lax.fori_loop(0, (R - 2) // 2, _pair, 0)
    _drain(didx_a, sem_a)
    _fire(didx_a, sem_a, R - 1)
    _drain(didx_b, sem_b)
    _drain(didx_a, sem_a)
    plsc.subcore_barrier()

    lane0 = lax.iota(jnp.int32, L) == 0

    def _inv(j, _):
        r = row0 + j * L
        pltpu.sync_copy(acc_sh.at[pl.ds(r, L)], chunk_v)
        for i in range(L):
            row = chunk_v[i, pl.ds(0, L)]
            iv_row = 1.0 / jnp.maximum(row, 1.0)
            plsc.store_scatter(out_v, [jnp.full((L,), i, jnp.int32)],
                               iv_row, mask=lane0)
        pltpu.sync_copy(out_v, invd_hbm.at[pl.ds(c * ACC_ROWS + r, L)])
        return 0
    lax.fori_loop(0, TPT // L, _inv, 0)


_deg_call = pl.kernel(
    _deg_body,
    out_type=jax.ShapeDtypeStruct((NPAD,), jnp.float32),
    mesh=_MESH,
    compiler_params=_PARAMS,
    scratch_types=[
        pltpu.VMEM((K, G), jnp.int32),
        pltpu.VMEM((K, G), jnp.int32),
        pltpu.VMEM((G, L), jnp.float32),
        pltpu.VMEM((L, L), jnp.float32),
        pltpu.VMEM((L,), jnp.float32),
        pltpu.SemaphoreType.DMA,
        pltpu.SemaphoreType.DMA,
        pltpu.VMEM_SHARED((ACC_ROWS, L), jnp.float32),
    ],
)


def _conv_body(xlo_hbm, xhi_hbm, src_hbm, dst_hbm, invd_hbm,
               ylo_hbm, yhi_hbm,
               sidx_a, didx_a, rows_a, sidx_b, didx_b, rows_b,
               nrm_v, invd_v, gsem_a, gsem_b, ssem_a, ssem_b, acc_sh):
    c = lax.axis_index("c")
    s = lax.axis_index("s")
    half_base = c * HALF
    row0 = s * TPT
    trash = HALF + s
    gbase0 = s * GPT
    zero16 = jnp.zeros((L,), jnp.float32)

    for x_hbm, y_hbm in ((xlo_hbm, ylo_hbm), (xhi_hbm, yhi_hbm)):
        def _zr(j, _):
            for k in range(DH // L):
                rows_a[0, j, pl.ds(k * L, L)] = zero16
            return 0
        lax.fori_loop(0, G, _zr, 0)
        zrows = rows_a.at[0]

        def _zc(j, _):
            pltpu.sync_copy(zrows, acc_sh.at[pl.ds(row0 + j * G, G)])
            return 0
        lax.fori_loop(0, TPT // G, _zc, 0)
        pltpu.sync_copy(zrows.at[pl.ds(0, TPT - (TPT // G) * G)],
                        acc_sh.at[pl.ds(row0 + (TPT // G) * G,
                                        TPT - (TPT // G) * G)])
        plsc.subcore_barrier()

        def _fire_g(sidx, didx, rows, gsem, r):
            gb = gbase0 + r * K
            pltpu.sync_copy(src_hbm.at[pl.ds(gb, K)], sidx)
            pltpu.sync_copy(dst_hbm.at[pl.ds(gb, K)], didx)
            for j in range(K):
                _remap_src(sidx, j)
                _remap_dst(didx, j, half_base, trash)
                pltpu.async_copy(x_hbm.at[sidx.at[j]], rows.at[j], gsem)

        def _drain_g(sidx, rows, gsem):
            for j in range(K):
                pltpu.make_async_copy(x_hbm.at[sidx.at[j]], rows.at[j],
                                      gsem).wait()

        def _fire_s(didx, rows, ssem):
            def _one(j, _):
                pltpu.async_copy(rows.at[j], acc_sh.at[didx.at[j]], ssem,
                                 add=True)
                return 0
            lax.fori_loop(0, K, _one, 0)

        def _drain_s(rows, ssem):
            for j in range(K):
                pltpu.make_async_copy(x_hbm.at[pl.ds(0, G)], rows.at[j],
                                      ssem).wait()

        _fire_g(sidx_a, didx_a, rows_a, gsem_a, 0)
        _fire_g(sidx_b, didx_b, rows_b, gsem_b, 1)
        _drain_g(sidx_a, rows_a, gsem_a)
        _fire_s(didx_a, rows_a, ssem_a)

        def _pair(i, _):
            r = 2 + 2 * i
            _drain_s(rows_a, ssem_a)
            _fire_g(sidx_a, didx_a, rows_a, gsem_a, r)
            _drain_g(sidx_b, rows_b, gsem_b)
            _fire_s(didx_b, rows_b, ssem_b)
            _drain_s(rows_b, ssem_b)
            _fire_g(sidx_b, didx_b, rows_b, gsem_b, r + 1)
            _drain_g(sidx_a, rows_a, gsem_a)
            _fire_s(didx_a, rows_a, ssem_a)
            return 0
        lax.fori_loop(0, (R - 3) // 2, _pair, 0)
        _drain_s(rows_a, ssem_a)
        _fire_g(sidx_a, didx_a, rows_a, gsem_a, R - 1)
        _drain_g(sidx_b, rows_b, gsem_b)
        _fire_s(didx_b, rows_b, ssem_b)
        _drain_s(rows_b, ssem_b)
        _drain_g(sidx_a, rows_a, gsem_a)
        _fire_s(didx_a, rows_a, ssem_a)
        _drain_s(rows_a, ssem_a)
        plsc.subcore_barrier()

        def _nrm(j, _):
            r = row0 + j * L
            pltpu.sync_copy(acc_sh.at[pl.ds(r, L)], nrm_v)
            pltpu.sync_copy(invd_hbm.at[pl.ds(c * ACC_ROWS + r, L)], invd_v)
            ivv = invd_v[pl.ds(0, L)]
            for i in range(L):
                iv = jnp.full((L,), ivv[i], jnp.float32)
                for k in range(DH // L):
                    sl = pl.ds(k * L, L)
                    nrm_v[i, sl] = nrm_v[i, sl] * iv
            pltpu.sync_copy(nrm_v, y_hbm.at[pl.ds(c * ACC_ROWS + r, L)])
            return 0
        lax.fori_loop(0, TPT // L, _nrm, 0)
        plsc.subcore_barrier()


_conv_call = pl.kernel(
    _conv_body,
    out_type=(jax.ShapeDtypeStruct((NPAD, DH), jnp.float32),
              jax.ShapeDtypeStruct((NPAD, DH), jnp.float32)),
    mesh=_MESH,
    compiler_params=_PARAMS,
    scratch_types=[
        pltpu.VMEM((K, G), jnp.int32),
        pltpu.VMEM((K, G), jnp.int32),
        pltpu.VMEM((K, G, DH), jnp.float32),
        pltpu.VMEM((K, G), jnp.int32),
        pltpu.VMEM((K, G), jnp.int32),
        pltpu.VMEM((K, G, DH), jnp.float32),
        pltpu.VMEM((L, DH), jnp.float32),
        pltpu.VMEM((L,), jnp.float32),
        pltpu.SemaphoreType.DMA,
        pltpu.SemaphoreType.DMA,
        pltpu.SemaphoreType.DMA,
        pltpu.SemaphoreType.DMA,
        pltpu.VMEM_SHARED((ACC_ROWS, DH), jnp.float32),
    ],
)


def _final_body(x0l, x0h, y1l, y1h, y2l, y2h, y3l, y3h, out_hbm,
                al_v, bl_v, cl_v, dl_v, ah_v, bh_v, ch_v, dh_v, out_v, sem):
    c = lax.axis_index("c")
    s = lax.axis_index("s")
    w = c * NS + s
    r0 = w * TPT_F
    nch = jnp.maximum(jnp.minimum(TPT_F, N_NODE - r0), 0) // RC_F

    def _chunk(j, _):
        rlog = r0 + j * RC_F
        rphy = rlog + jnp.where(rlog >= HALF, PAD_OFF, 0)
        srcs = ((x0l, al_v), (y1l, bl_v), (y2l, cl_v), (y3l, dl_v),
                (x0h, ah_v), (y1h, bh_v), (y2h, ch_v), (y3h, dh_v))
        for t, buf in srcs:
            pltpu.async_copy(t.at[pl.ds(rphy, RC_F)], buf, sem)
        for t, buf in srcs:
            pltpu.make_async_copy(t.at[pl.ds(rphy, RC_F)], buf, sem).wait()
        for i in range(RC_F):
            for k in range(DH // L):
                sl = pl.ds(k * L, L)
                out_v[i, pl.ds(k * L, L)] = (
                    al_v[i, sl] + bl_v[i, sl] + cl_v[i, sl] + dl_v[i, sl]
                ) * ALPHA
                out_v[i, pl.ds(DH + k * L, L)] = (
                    ah_v[i, sl] + bh_v[i, sl] + ch_v[i, sl] + dh_v[i, sl]
                ) * ALPHA
        pltpu.sync_copy(out_v, out_hbm.at[pl.ds(rlog, RC_F)])
        return 0
    lax.fori_loop(0, nch, _chunk, 0)


_final_call = pl.kernel(
    _final_body,
    out_type=jax.ShapeDtypeStruct((N_NODE, DIM), jnp.float32),
    mesh=_MESH,
    compiler_params=_PARAMS,
    scratch_types=[
        pltpu.VMEM((RC_F, DH), jnp.float32),
        pltpu.VMEM((RC_F, DH), jnp.float32),
        pltpu.VMEM((RC_F, DH), jnp.float32),
        pltpu.VMEM((RC_F, DH), jnp.float32),
        pltpu.VMEM((RC_F, DH), jnp.float32),
        pltpu.VMEM((RC_F, DH), jnp.float32),
        pltpu.VMEM((RC_F, DH), jnp.float32),
        pltpu.VMEM((RC_F, DH), jnp.float32),
        pltpu.VMEM((RC_F, DIM), jnp.float32),
        pltpu.SemaphoreType.DMA,
    ],
)


def _pad(t):
    z = jnp.zeros((PAD_OFF, DH), jnp.float32)
    return jnp.concatenate([t[:HALF], z, t[HALF:], z], axis=0)


def _pad_edges(ei):
    npad = EPAD - E
    src = jnp.concatenate([ei[0], jnp.zeros((npad,), jnp.int32)])
    dst = jnp.concatenate([ei[1], jnp.full((npad,), -1, jnp.int32)])
    return src.reshape(EROWS, G), dst.reshape(EROWS, G)


def kernel(x_recipe, usr_emb, rcp_emb, edge_weight_usr_rcp,
           edge_weight_rcp_usr, edge_index_usr_rcp, edge_index_rcp_usr):
    del edge_weight_usr_rcp, edge_weight_rcp_usr
    src_ur, dst_ur = _pad_edges(edge_index_usr_rcp)
    src_ru, dst_ru = _pad_edges(edge_index_rcp_usr)

    usr0 = (_pad(usr_emb[:, :DH]), _pad(usr_emb[:, DH:]))
    rcp_x = jnp.concatenate([rcp_emb, x_recipe], axis=1)
    rcp0 = (_pad(rcp_x[:, :DH]), _pad(rcp_x[:, DH:]))

    invd_rcp = _deg_call(dst_ur)
    invd_usr = _deg_call(dst_ru)

    x_u = usr0
    ys_r = []
    ys_u = []
    for _ in range(N_LAYERS):
        x_r = _conv_call(x_u[0], x_u[1], src_ur, dst_ur, invd_rcp)
        ys_r.append(x_r)
        x_u = _conv_call(x_r[0], x_r[1], src_ru, dst_ru, invd_usr)
        ys_u.append(x_u)

    usr_out = _final_call(usr0[0], usr0[1], ys_u[0][0], ys_u[0][1],
                          ys_u[1][0], ys_u[1][1], ys_u[2][0], ys_u[2][1])
    rec_out = _final_call(rcp0[0], rcp0[1], ys_r[0][0], ys_r[0][1],
                          ys_r[1][0], ys_r[1][1], ys_r[2][0], ys_r[2][1])
    return (usr_out, rec_out)


# speedup vs baseline: 6.4559x; 1.0794x over previous
import jax
import jax.numpy as jnp
from jax import lax
from jax.experimental import pallas as pl
from jax.experimental.pallas import tpu as pltpu
from jax.experimental.pallas import tpu_sc as plsc

N_NODE = 50000
DIM = 64
DH = 32
E = 800000
N_LAYERS = 3
ALPHA = 1.0 / (N_LAYERS + 1)

NC = 2
NS = 16
L = 16

HALF = N_NODE // 2
TPT = 1568
ACC_ROWS = NS * TPT
NPAD = NC * ACC_ROWS
PAD_OFF = ACC_ROWS - HALF

G = 128
K = 8
EPAD = 512 * TPT
GPT = EPAD // NS // G
R = GPT // K
EROWS = EPAD // G

TPT_F = 1600
RC_F = 40

_MESH = plsc.VectorSubcoreMesh(core_axis_name="c", subcore_axis_name="s")
_PARAMS = pltpu.CompilerParams(needs_layout_passes=False,
                               use_tc_tiling_on_sc=False)


def _remap_dst(dref, j, half_base, trash, src=None):
    src = dref if src is None else src
    for k in range(G // L):
        sl = pl.ds(k * L, L)
        di = src[j, sl]
        dl = di - half_base
        inb = (dl >= 0) & (dl < HALF)
        dref[j, sl] = jnp.where(inb, dl, trash)


def _remap_src(sref, j, src=None):
    src = sref if src is None else src
    for k in range(G // L):
        sl = pl.ds(k * L, L)
        si = src[j, sl]
        sref[j, sl] = jnp.where(si >= HALF, si + PAD_OFF, si)


def _deg_body(dst_hbm, invd_hbm, didx_a, didx_b, ones_v, chunk_v, out_v,
              sem_a, sem_b, acc_sh):
    c = lax.axis_index("c")
    s = lax.axis_index("s")
    half_base = c * HALF
    row0 = s * TPT
    trash = HALF + s
    gbase0 = s * GPT
    zero16 = jnp.zeros((L,), jnp.float32)

    def _zr(j, _):
        ones_v[j, pl.ds(0, L)] = zero16
        return 0
    lax.fori_loop(0, G, _zr, 0)

    def _zc(j, _):
        pltpu.sync_copy(ones_v, acc_sh.at[pl.ds(row0 + j * G, G)])
        return 0
    lax.fori_loop(0, TPT // G, _zc, 0)
    pltpu.sync_copy(ones_v.at[pl.ds(0, TPT - (TPT // G) * G)],
                    acc_sh.at[pl.ds(row0 + (TPT // G) * G,
                                    TPT - (TPT // G) * G)])

    patt = jnp.where(lax.iota(jnp.int32, L) == 0, 1.0, 0.0).astype(jnp.float32)

    def _sp(j, _):
        ones_v[j, pl.ds(0, L)] = patt
        return 0
    lax.fori_loop(0, G, _sp, 0)
    plsc.subcore_barrier()

    def _fire(didx, sem, r):
        gb = gbase0 + r * K
        pltpu.sync_copy(dst_hbm.at[pl.ds(gb, K)], didx)
        for j in range(K):
            _remap_dst(didx, j, half_base, trash)
            pltpu.async_copy(ones_v, acc_sh.at[didx.at[j]], sem, add=True)

    def _drain(didx, sem):
        for j in range(K):
            pltpu.make_async_copy(ones_v, acc_sh.at[didx.at[j]], sem).wait()

    _fire(didx_a, sem_a, 0)
    _fire(didx_b, sem_b, 1)

    def _pair(i, _):
        r = 2 + 2 * i
        _drain(didx_a, sem_a)
        _fire(didx_a, sem_a, r)
        _drain(didx_b, sem_b)
        _fire(didx_b, sem_b, r + 1)
        return 0
    lax.fori_loop(0, (R - 2) // 2, _pair, 0)
    _drain(didx_a, sem_a)
    _fire(didx_a, sem_a, R - 1)
    _drain(didx_b, sem_b)
    _drain(didx_a, sem_a)
    plsc.subcore_barrier()

    lane0 = lax.iota(jnp.int32, L) == 0

    def _inv(j, _):
        r = row0 + j * L
        pltpu.sync_copy(acc_sh.at[pl.ds(r, L)], chunk_v)
        for i in range(L):
            row = chunk_v[i, pl.ds(0, L)]
            iv_row = 1.0 / jnp.maximum(row, 1.0)
            plsc.store_scatter(out_v, [jnp.full((L,), i, jnp.int32)],
                               iv_row, mask=lane0)
        pltpu.sync_copy(out_v, invd_hbm.at[pl.ds(c * ACC_ROWS + r, L)])
        return 0
    lax.fori_loop(0, TPT // L, _inv, 0)


_deg_call = pl.kernel(
    _deg_body,
    out_type=jax.ShapeDtypeStruct((NPAD,), jnp.float32),
    mesh=_MESH,
    compiler_params=_PARAMS,
    scratch_types=[
        pltpu.VMEM((K, G), jnp.int32),
        pltpu.VMEM((K, G), jnp.int32),
        pltpu.VMEM((G, L), jnp.float32),
        pltpu.VMEM((L, L), jnp.float32),
        pltpu.VMEM((L,), jnp.float32),
        pltpu.SemaphoreType.DMA,
        pltpu.SemaphoreType.DMA,
        pltpu.VMEM_SHARED((ACC_ROWS, L), jnp.float32),
    ],
)


def _conv_body(xlo_hbm, xhi_hbm, src_hbm, dst_hbm, invd_hbm,
               ylo_hbm, yhi_hbm,
               sidx_a, didx_a, rows_a, sidx_b, didx_b, rows_b,
               pidx_sa, pidx_da, pidx_sb, pidx_db,
               nrm_v, invd_v, gsem_a, gsem_b, ssem_a, ssem_b,
               isem_a, isem_b, acc_sh):
    c = lax.axis_index("c")
    s = lax.axis_index("s")
    half_base = c * HALF
    row0 = s * TPT
    trash = HALF + s
    gbase0 = s * GPT
    zero16 = jnp.zeros((L,), jnp.float32)

    for x_hbm, y_hbm in ((xlo_hbm, ylo_hbm), (xhi_hbm, yhi_hbm)):
        def _zr(j, _):
            for k in range(DH // L):
                rows_a[0, j, pl.ds(k * L, L)] = zero16
            return 0
        lax.fori_loop(0, G, _zr, 0)
        zrows = rows_a.at[0]

        def _zc(j, _):
            pltpu.sync_copy(zrows, acc_sh.at[pl.ds(row0 + j * G, G)])
            return 0
        lax.fori_loop(0, TPT // G, _zc, 0)
        pltpu.sync_copy(zrows.at[pl.ds(0, TPT - (TPT // G) * G)],
                        acc_sh.at[pl.ds(row0 + (TPT // G) * G,
                                        TPT - (TPT // G) * G)])
        plsc.subcore_barrier()

        def _fire_i(ps, pd, isem, r):
            gb = gbase0 + r * K
            pltpu.async_copy(src_hbm.at[pl.ds(gb, K)], ps, isem)
            pltpu.async_copy(dst_hbm.at[pl.ds(gb, K)], pd, isem)

        def _finish_i(ps, pd, isem, r):
            pltpu.make_async_copy(src_hbm.at[pl.ds(gbase0 + r * K, K)],
                                  ps, isem).wait()
            pltpu.make_async_copy(dst_hbm.at[pl.ds(gbase0 + r * K, K)],
                                  pd, isem).wait()

        def _fire_g(sidx, didx, rows, gsem, ps, pd, isem, r):
            _finish_i(ps, pd, isem, r)
            for j in range(K):
                _remap_src(sidx, j, src=ps)
                _remap_dst(didx, j, half_base, trash, src=pd)
            nxt = r + NC
            @pl.when(nxt < R)
            def _():
                _fire_i(ps, pd, isem, nxt)
            for j in range(K):
                pltpu.async_copy(x_hbm.at[sidx.at[j]], rows.at[j], gsem)

        def _drain_g(sidx, rows, gsem):
            for j in range(K):
                pltpu.make_async_copy(x_hbm.at[sidx.at[j]], rows.at[j],
                                      gsem).wait()

        def _fire_s(didx, rows, ssem):
            def _one(j, _):
                pltpu.async_copy(rows.at[j], acc_sh.at[didx.at[j]], ssem,
                                 add=True)
                return 0
            lax.fori_loop(0, K, _one, 0)

        def _drain_s(rows, ssem):
            for j in range(K):
                pltpu.make_async_copy(x_hbm.at[pl.ds(0, G)], rows.at[j],
                                      ssem).wait()

        _fire_i(pidx_sa, pidx_da, isem_a, 0)
        _fire_i(pidx_sb, pidx_db, isem_b, 1)
        _fire_g(sidx_a, didx_a, rows_a, gsem_a, pidx_sa, pidx_da, isem_a, 0)
        _fire_g(sidx_b, didx_b, rows_b, gsem_b, pidx_sb, pidx_db, isem_b, 1)
        _drain_g(sidx_a, rows_a, gsem_a)
        _fire_s(didx_a, rows_a, ssem_a)

        def _pair(i, _):
            r = 2 + 2 * i
            _drain_s(rows_a, ssem_a)
            _fire_g(sidx_a, didx_a, rows_a, gsem_a,
                    pidx_sa, pidx_da, isem_a, r)
            _drain_g(sidx_b, rows_b, gsem_b)
            _fire_s(didx_b, rows_b, ssem_b)
            _drain_s(rows_b, ssem_b)
            _fire_g(sidx_b, didx_b, rows_b, gsem_b,
                    pidx_sb, pidx_db, isem_b, r + 1)
            _drain_g(sidx_a, rows_a, gsem_a)
            _fire_s(didx_a, rows_a, ssem_a)
            return 0
        lax.fori_loop(0, (R - 3) // 2, _pair, 0)
        _drain_s(rows_a, ssem_a)
        _fire_g(sidx_a, didx_a, rows_a, gsem_a, pidx_sa, pidx_da, isem_a,
                R - 1)
        _drain_g(sidx_b, rows_b, gsem_b)
        _fire_s(didx_b, rows_b, ssem_b)
        _drain_s(rows_b, ssem_b)
        _drain_g(sidx_a, rows_a, gsem_a)
        _fire_s(didx_a, rows_a, ssem_a)
        _drain_s(rows_a, ssem_a)
        plsc.subcore_barrier()

        def _nrm(j, _):
            r = row0 + j * L
            pltpu.sync_copy(acc_sh.at[pl.ds(r, L)], nrm_v)
            pltpu.sync_copy(invd_hbm.at[pl.ds(c * ACC_ROWS + r, L)], invd_v)
            ivv = invd_v[pl.ds(0, L)]
            for i in range(L):
                iv = jnp.full((L,), ivv[i], jnp.float32)
                for k in range(DH // L):
                    sl = pl.ds(k * L, L)
                    nrm_v[i, sl] = nrm_v[i, sl] * iv
            pltpu.sync_copy(nrm_v, y_hbm.at[pl.ds(c * ACC_ROWS + r, L)])
            return 0
        lax.fori_loop(0, TPT // L, _nrm, 0)
        plsc.subcore_barrier()


_conv_call = pl.kernel(
    _conv_body,
    out_type=(jax.ShapeDtypeStruct((NPAD, DH), jnp.float32),
              jax.ShapeDtypeStruct((NPAD, DH), jnp.float32)),
    mesh=_MESH,
    compiler_params=_PARAMS,
    scratch_types=[
        pltpu.VMEM((K, G), jnp.int32),
        pltpu.VMEM((K, G), jnp.int32),
        pltpu.VMEM((K, G, DH), jnp.float32),
        pltpu.VMEM((K, G), jnp.int32),
        pltpu.VMEM((K, G), jnp.int32),
        pltpu.VMEM((K, G, DH), jnp.float32),
        pltpu.VMEM((K, G), jnp.int32),
        pltpu.VMEM((K, G), jnp.int32),
        pltpu.VMEM((K, G), jnp.int32),
        pltpu.VMEM((K, G), jnp.int32),
        pltpu.VMEM((L, DH), jnp.float32),
        pltpu.VMEM((L,), jnp.float32),
        pltpu.SemaphoreType.DMA,
        pltpu.SemaphoreType.DMA,
        pltpu.SemaphoreType.DMA,
        pltpu.SemaphoreType.DMA,
        pltpu.SemaphoreType.DMA,
        pltpu.SemaphoreType.DMA,
        pltpu.VMEM_SHARED((ACC_ROWS, DH), jnp.float32),
    ],
)


def _final_body(x0l, x0h, y1l, y1h, y2l, y2h, y3l, y3h, out_hbm,
                al_v, bl_v, cl_v, dl_v, ah_v, bh_v, ch_v, dh_v, out_v, sem):
    c = lax.axis_index("c")
    s = lax.axis_index("s")
    w = c * NS + s
    r0 = w * TPT_F
    nch = jnp.maximum(jnp.minimum(TPT_F, N_NODE - r0), 0) // RC_F

    def _chunk(j, _):
        rlog = r0 + j * RC_F
        rphy = rlog + jnp.where(rlog >= HALF, PAD_OFF, 0)
        srcs = ((x0l, al_v), (y1l, bl_v), (y2l, cl_v), (y3l, dl_v),
                (x0h, ah_v), (y1h, bh_v), (y2h, ch_v), (y3h, dh_v))
        for t, buf in srcs:
            pltpu.async_copy(t.at[pl.ds(rphy, RC_F)], buf, sem)
        for t, buf in srcs:
            pltpu.make_async_copy(t.at[pl.ds(rphy, RC_F)], buf, sem).wait()
        for i in range(RC_F):
            for k in range(DH // L):
                sl = pl.ds(k * L, L)
                out_v[i, pl.ds(k * L, L)] = (
                    al_v[i, sl] + bl_v[i, sl] + cl_v[i, sl] + dl_v[i, sl]
                ) * ALPHA
                out_v[i, pl.ds(DH + k * L, L)] = (
                    ah_v[i, sl] + bh_v[i, sl] + ch_v[i, sl] + dh_v[i, sl]
                ) * ALPHA
        pltpu.sync_copy(out_v, out_hbm.at[pl.ds(rlog, RC_F)])
        return 0
    lax.fori_loop(0, nch, _chunk, 0)


_final_call = pl.kernel(
    _final_body,
    out_type=jax.ShapeDtypeStruct((N_NODE, DIM), jnp.float32),
    mesh=_MESH,
    compiler_params=_PARAMS,
    scratch_types=[
        pltpu.VMEM((RC_F, DH), jnp.float32),
        pltpu.VMEM((RC_F, DH), jnp.float32),
        pltpu.VMEM((RC_F, DH), jnp.float32),
        pltpu.VMEM((RC_F, DH), jnp.float32),
        pltpu.VMEM((RC_F, DH), jnp.float32),
        pltpu.VMEM((RC_F, DH), jnp.float32),
        pltpu.VMEM((RC_F, DH), jnp.float32),
        pltpu.VMEM((RC_F, DH), jnp.float32),
        pltpu.VMEM((RC_F, DIM), jnp.float32),
        pltpu.SemaphoreType.DMA,
    ],
)


def _pad(t):
    z = jnp.zeros((PAD_OFF, DH), jnp.float32)
    return jnp.concatenate([t[:HALF], z, t[HALF:], z], axis=0)


def _pad_edges(ei):
    npad = EPAD - E
    src = jnp.concatenate([ei[0], jnp.zeros((npad,), jnp.int32)])
    dst = jnp.concatenate([ei[1], jnp.full((npad,), -1, jnp.int32)])
    return src.reshape(EROWS, G), dst.reshape(EROWS, G)


def kernel(x_recipe, usr_emb, rcp_emb, edge_weight_usr_rcp,
           edge_weight_rcp_usr, edge_index_usr_rcp, edge_index_rcp_usr):
    del edge_weight_usr_rcp, edge_weight_rcp_usr
    src_ur, dst_ur = _pad_edges(edge_index_usr_rcp)
    src_ru, dst_ru = _pad_edges(edge_index_rcp_usr)

    usr0 = (_pad(usr_emb[:, :DH]), _pad(usr_emb[:, DH:]))
    rcp_x = jnp.concatenate([rcp_emb, x_recipe], axis=1)
    rcp0 = (_pad(rcp_x[:, :DH]), _pad(rcp_x[:, DH:]))

    invd_rcp = _deg_call(dst_ur)
    invd_usr = _deg_call(dst_ru)

    x_u = usr0
    ys_r = []
    ys_u = []
    for _ in range(N_LAYERS):
        x_r = _conv_call(x_u[0], x_u[1], src_ur, dst_ur, invd_rcp)
        ys_r.append(x_r)
        x_u = _conv_call(x_r[0], x_r[1], src_ru, dst_ru, invd_usr)
        ys_u.append(x_u)

    usr_out = _final_call(usr0[0], usr0[1], ys_u[0][0], ys_u[0][1],
                          ys_u[1][0], ys_u[1][1], ys_u[2][0], ys_u[2][1])
    rec_out = _final_call(rcp0[0], rcp0[1], ys_r[0][0], ys_r[0][1],
                          ys_r[1][0], ys_r[1][1], ys_r[2][0], ys_r[2][1])
    return (usr_out, rec_out)


# speedup vs baseline: 6.5713x; 1.0179x over previous
import jax
import jax.numpy as jnp
from jax import lax
from jax.experimental import pallas as pl
from jax.experimental.pallas import tpu as pltpu
from jax.experimental.pallas import tpu_sc as plsc

N_NODE = 50000
DIM = 64
DH = 32
E = 800000
N_LAYERS = 3
ALPHA = 1.0 / (N_LAYERS + 1)

NC = 2
NS = 16
L = 16

HALF = N_NODE // 2
TPT = 1568
ACC_ROWS = NS * TPT
NPAD = NC * ACC_ROWS
PAD_OFF = ACC_ROWS - HALF

G = 128
K = 8
EPAD = 512 * TPT
GPT = EPAD // NS // G
R = GPT // K
EROWS = EPAD // G

TPT_F = 1600
RC_F = 40

_MESH = plsc.VectorSubcoreMesh(core_axis_name="c", subcore_axis_name="s")
_PARAMS = pltpu.CompilerParams(needs_layout_passes=False,
                               use_tc_tiling_on_sc=False)


def _remap_dst(dref, j, half_base, trash, src=None):
    src = dref if src is None else src
    for k in range(G // L):
        sl = pl.ds(k * L, L)
        di = src[j, sl]
        dl = di - half_base
        inb = (dl >= 0) & (dl < HALF)
        dref[j, sl] = jnp.where(inb, dl, trash)


def _remap_src(sref, j, src=None):
    src = sref if src is None else src
    for k in range(G // L):
        sl = pl.ds(k * L, L)
        si = src[j, sl]
        sref[j, sl] = jnp.where(si >= HALF, si + PAD_OFF, si)


def _deg_body(dst_hbm, invd_hbm, didx_a, didx_b, ones_v, chunk_v, out_v,
              sem_a, sem_b, acc_sh):
    c = lax.axis_index("c")
    s = lax.axis_index("s")
    half_base = c * HALF
    row0 = s * TPT
    trash = HALF + s
    gbase0 = s * GPT
    zero16 = jnp.zeros((L,), jnp.float32)

    def _zr(j, _):
        ones_v[j, pl.ds(0, L)] = zero16
        return 0
    lax.fori_loop(0, G, _zr, 0)

    def _zc(j, _):
        pltpu.sync_copy(ones_v, acc_sh.at[pl.ds(row0 + j * G, G)])
        return 0
    lax.fori_loop(0, TPT // G, _zc, 0)
    pltpu.sync_copy(ones_v.at[pl.ds(0, TPT - (TPT // G) * G)],
                    acc_sh.at[pl.ds(row0 + (TPT // G) * G,
                                    TPT - (TPT // G) * G)])

    patt = jnp.where(lax.iota(jnp.int32, L) == 0, 1.0, 0.0).astype(jnp.float32)

    def _sp(j, _):
        ones_v[j, pl.ds(0, L)] = patt
        return 0
    lax.fori_loop(0, G, _sp, 0)
    plsc.subcore_barrier()

    def _fire(didx, sem, r):
        gb = gbase0 + r * K
        pltpu.sync_copy(dst_hbm.at[pl.ds(gb, K)], didx)
        for j in range(K):
            _remap_dst(didx, j, half_base, trash)
            pltpu.async_copy(ones_v, acc_sh.at[didx.at[j]], sem, add=True)

    def _drain(didx, sem):
        for j in range(K):
            pltpu.make_async_copy(ones_v, acc_sh.at[didx.at[j]], sem).wait()

    _fire(didx_a, sem_a, 0)
    _fire(didx_b, sem_b, 1)

    def _pair(i, _):
        r = 2 + 2 * i
        _drain(didx_a, sem_a)
        _fire(didx_a, sem_a, r)
        _drain(didx_b, sem_b)
        _fire(didx_b, sem_b, r + 1)
        return 0
    lax.fori_loop(0, (R - 2) // 2, _pair, 0)
    _drain(didx_a, sem_a)
    _fire(didx_a, sem_a, R - 1)
    _drain(didx_b, sem_b)
    _drain(didx_a, sem_a)
    plsc.subcore_barrier()

    lane0 = lax.iota(jnp.int32, L) == 0

    def _inv(j, _):
        r = row0 + j * L
        pltpu.sync_copy(acc_sh.at[pl.ds(r, L)], chunk_v)
        for i in range(L):
            row = chunk_v[i, pl.ds(0, L)]
            iv_row = 1.0 / jnp.maximum(row, 1.0)
            plsc.store_scatter(out_v, [jnp.full((L,), i, jnp.int32)],
                               iv_row, mask=lane0)
        pltpu.sync_copy(out_v, invd_hbm.at[pl.ds(c * ACC_ROWS + r, L)])
        return 0
    lax.fori_loop(0, TPT // L, _inv, 0)


_deg_call = pl.kernel(
    _deg_body,
    out_type=jax.ShapeDtypeStruct((NPAD,), jnp.float32),
    mesh=_MESH,
    compiler_params=_PARAMS,
    scratch_types=[
        pltpu.VMEM((K, G), jnp.int32),
        pltpu.VMEM((K, G), jnp.int32),
        pltpu.VMEM((G, L), jnp.float32),
        pltpu.VMEM((L, L), jnp.float32),
        pltpu.VMEM((L,), jnp.float32),
        pltpu.SemaphoreType.DMA,
        pltpu.SemaphoreType.DMA,
        pltpu.VMEM_SHARED((ACC_ROWS, L), jnp.float32),
    ],
)


def _conv_body(xlo_hbm, xhi_hbm, src_hbm, dst_hbm, invd_hbm,
               ylo_hbm, yhi_hbm,
               didx_a, rows_a, didx_b, rows_b,
               pidx_sa, pidx_da, pidx_sb, pidx_db,
               nrm_v, invd_v, gsem_a, gsem_b, ssem_a, ssem_b,
               isem_a, isem_b, acc_sh):
    c = lax.axis_index("c")
    s = lax.axis_index("s")
    half_base = c * HALF
    row0 = s * TPT
    trash = HALF + s
    gbase0 = s * GPT
    zero16 = jnp.zeros((L,), jnp.float32)

    for x_hbm, y_hbm in ((xlo_hbm, ylo_hbm), (xhi_hbm, yhi_hbm)):
        def _zr(j, _):
            for k in range(DH // L):
                rows_a[0, j, pl.ds(k * L, L)] = zero16
            return 0
        lax.fori_loop(0, G, _zr, 0)
        zrows = rows_a.at[0]

        def _zc(j, _):
            pltpu.sync_copy(zrows, acc_sh.at[pl.ds(row0 + j * G, G)])
            return 0
        lax.fori_loop(0, TPT // G, _zc, 0)
        pltpu.sync_copy(zrows.at[pl.ds(0, TPT - (TPT // G) * G)],
                        acc_sh.at[pl.ds(row0 + (TPT // G) * G,
                                        TPT - (TPT // G) * G)])
        plsc.subcore_barrier()

        def _fire_i(ps, pd, isem, r):
            gb = gbase0 + r * K
            pltpu.async_copy(src_hbm.at[pl.ds(gb, K)], ps, isem)
            pltpu.async_copy(dst_hbm.at[pl.ds(gb, K)], pd, isem)

        def _finish_i(ps, pd, isem, r):
            pltpu.make_async_copy(src_hbm.at[pl.ds(gbase0 + r * K, K)],
                                  ps, isem).wait()
            pltpu.make_async_copy(dst_hbm.at[pl.ds(gbase0 + r * K, K)],
                                  pd, isem).wait()

        def _fire_g(didx, rows, gsem, ps, pd, isem, r):
            _finish_i(ps, pd, isem, r)
            for j in range(K):
                _remap_dst(didx, j, half_base, trash, src=pd)
            nxt = r + 2
            @pl.when(nxt < R)
            def _():
                pltpu.async_copy(dst_hbm.at[pl.ds(gbase0 + nxt * K, K)],
                                 pd, isem)
            for j in range(K):
                pltpu.async_copy(x_hbm.at[ps.at[j]], rows.at[j], gsem)

        def _drain_g(ps, rows, gsem, isem, r):
            for j in range(K):
                pltpu.make_async_copy(x_hbm.at[ps.at[j]], rows.at[j],
                                      gsem).wait()
            nxt = r + 2
            @pl.when(nxt < R)
            def _():
                pltpu.async_copy(src_hbm.at[pl.ds(gbase0 + nxt * K, K)],
                                 ps, isem)

        def _fire_s(didx, rows, ssem):
            def _one(j, _):
                pltpu.async_copy(rows.at[j], acc_sh.at[didx.at[j]], ssem,
                                 add=True)
                return 0
            lax.fori_loop(0, K, _one, 0)

        def _drain_s(rows, ssem):
            for j in range(K):
                pltpu.make_async_copy(x_hbm.at[pl.ds(0, G)], rows.at[j],
                                      ssem).wait()

        _fire_i(pidx_sa, pidx_da, isem_a, 0)
        _fire_i(pidx_sb, pidx_db, isem_b, 1)
        _fire_g(didx_a, rows_a, gsem_a, pidx_sa, pidx_da, isem_a, 0)
        _fire_g(didx_b, rows_b, gsem_b, pidx_sb, pidx_db, isem_b, 1)
        _drain_g(pidx_sa, rows_a, gsem_a, isem_a, 0)
        _fire_s(didx_a, rows_a, ssem_a)

        def _pair(i, _):
            r = 2 + 2 * i
            _drain_s(rows_a, ssem_a)
            _fire_g(didx_a, rows_a, gsem_a, pidx_sa, pidx_da, isem_a, r)
            _drain_g(pidx_sb, rows_b, gsem_b, isem_b, r - 1)
            _fire_s(didx_b, rows_b, ssem_b)
            _drain_s(rows_b, ssem_b)
            _fire_g(didx_b, rows_b, gsem_b, pidx_sb, pidx_db, isem_b, r + 1)
            _drain_g(pidx_sa, rows_a, gsem_a, isem_a, r)
            _fire_s(didx_a, rows_a, ssem_a)
            return 0
        lax.fori_loop(0, (R - 3) // 2, _pair, 0)
        _drain_s(rows_a, ssem_a)
        _fire_g(didx_a, rows_a, gsem_a, pidx_sa, pidx_da, isem_a, R - 1)
        _drain_g(pidx_sb, rows_b, gsem_b, isem_b, R - 2)
        _fire_s(didx_b, rows_b, ssem_b)
        _drain_s(rows_b, ssem_b)
        _drain_g(pidx_sa, rows_a, gsem_a, isem_a, R - 1)
        _fire_s(didx_a, rows_a, ssem_a)
        _drain_s(rows_a, ssem_a)
        plsc.subcore_barrier()

        def _nrm(j, _):
            r = row0 + j * L
            pltpu.sync_copy(acc_sh.at[pl.ds(r, L)], nrm_v)
            pltpu.sync_copy(invd_hbm.at[pl.ds(c * ACC_ROWS + r, L)], invd_v)
            ivv = invd_v[pl.ds(0, L)]
            for i in range(L):
                iv = jnp.full((L,), ivv[i], jnp.float32)
                for k in range(DH // L):
                    sl = pl.ds(k * L, L)
                    nrm_v[i, sl] = nrm_v[i, sl] * iv
            pltpu.sync_copy(nrm_v, y_hbm.at[pl.ds(c * ACC_ROWS + r, L)])
            return 0
        lax.fori_loop(0, TPT // L, _nrm, 0)
        plsc.subcore_barrier()


_conv_call = pl.kernel(
    _conv_body,
    out_type=(jax.ShapeDtypeStruct((NPAD, DH), jnp.float32),
              jax.ShapeDtypeStruct((NPAD, DH), jnp.float32)),
    mesh=_MESH,
    compiler_params=_PARAMS,
    scratch_types=[
        pltpu.VMEM((K, G), jnp.int32),
        pltpu.VMEM((K, G, DH), jnp.float32),
        pltpu.VMEM((K, G), jnp.int32),
        pltpu.VMEM((K, G, DH), jnp.float32),
        pltpu.VMEM((K, G), jnp.int32),
        pltpu.VMEM((K, G), jnp.int32),
        pltpu.VMEM((K, G), jnp.int32),
        pltpu.VMEM((K, G), jnp.int32),
        pltpu.VMEM((L, DH), jnp.float32),
        pltpu.VMEM((L,), jnp.float32),
        pltpu.SemaphoreType.DMA,
        pltpu.SemaphoreType.DMA,
        pltpu.SemaphoreType.DMA,
        pltpu.SemaphoreType.DMA,
        pltpu.SemaphoreType.DMA,
        pltpu.SemaphoreType.DMA,
        pltpu.VMEM_SHARED((ACC_ROWS, DH), jnp.float32),
    ],
)


def _final_body(x0l, x0h, y1l, y1h, y2l, y2h, y3l, y3h, out_hbm,
                al_v, bl_v, cl_v, dl_v, ah_v, bh_v, ch_v, dh_v, out_v, sem):
    c = lax.axis_index("c")
    s = lax.axis_index("s")
    w = c * NS + s
    r0 = w * TPT_F
    nch = jnp.maximum(jnp.minimum(TPT_F, N_NODE - r0), 0) // RC_F

    def _chunk(j, _):
        rlog = r0 + j * RC_F
        rphy = rlog + jnp.where(rlog >= HALF, PAD_OFF, 0)
        srcs = ((x0l, al_v), (y1l, bl_v), (y2l, cl_v), (y3l, dl_v),
                (x0h, ah_v), (y1h, bh_v), (y2h, ch_v), (y3h, dh_v))
        for t, buf in srcs:
            pltpu.async_copy(t.at[pl.ds(rphy, RC_F)], buf, sem)
        for t, buf in srcs:
            pltpu.make_async_copy(t.at[pl.ds(rphy, RC_F)], buf, sem).wait()
        for i in range(RC_F):
            for k in range(DH // L):
                sl = pl.ds(k * L, L)
                out_v[i, pl.ds(k * L, L)] = (
                    al_v[i, sl] + bl_v[i, sl] + cl_v[i, sl] + dl_v[i, sl]
                ) * ALPHA
                out_v[i, pl.ds(DH + k * L, L)] = (
                    ah_v[i, sl] + bh_v[i, sl] + ch_v[i, sl] + dh_v[i, sl]
                ) * ALPHA
        pltpu.sync_copy(out_v, out_hbm.at[pl.ds(rlog, RC_F)])
        return 0
    lax.fori_loop(0, nch, _chunk, 0)


_final_call = pl.kernel(
    _final_body,
    out_type=jax.ShapeDtypeStruct((N_NODE, DIM), jnp.float32),
    mesh=_MESH,
    compiler_params=_PARAMS,
    scratch_types=[
        pltpu.VMEM((RC_F, DH), jnp.float32),
        pltpu.VMEM((RC_F, DH), jnp.float32),
        pltpu.VMEM((RC_F, DH), jnp.float32),
        pltpu.VMEM((RC_F, DH), jnp.float32),
        pltpu.VMEM((RC_F, DH), jnp.float32),
        pltpu.VMEM((RC_F, DH), jnp.float32),
        pltpu.VMEM((RC_F, DH), jnp.float32),
        pltpu.VMEM((RC_F, DH), jnp.float32),
        pltpu.VMEM((RC_F, DIM), jnp.float32),
        pltpu.SemaphoreType.DMA,
    ],
)


def _pad(t):
    z = jnp.zeros((PAD_OFF, DH), jnp.float32)
    return jnp.concatenate([t[:HALF], z, t[HALF:], z], axis=0)


def _pad_edges(ei):
    npad = EPAD - E
    src = jnp.concatenate([ei[0], jnp.zeros((npad,), jnp.int32)])
    src = src + PAD_OFF * (src >= HALF).astype(jnp.int32)
    dst = jnp.concatenate([ei[1], jnp.full((npad,), -1, jnp.int32)])
    return src.reshape(EROWS, G), dst.reshape(EROWS, G)


def kernel(x_recipe, usr_emb, rcp_emb, edge_weight_usr_rcp,
           edge_weight_rcp_usr, edge_index_usr_rcp, edge_index_rcp_usr):
    del edge_weight_usr_rcp, edge_weight_rcp_usr
    src_ur, dst_ur = _pad_edges(edge_index_usr_rcp)
    src_ru, dst_ru = _pad_edges(edge_index_rcp_usr)

    usr0 = (_pad(usr_emb[:, :DH]), _pad(usr_emb[:, DH:]))
    rcp_x = jnp.concatenate([rcp_emb, x_recipe], axis=1)
    rcp0 = (_pad(rcp_x[:, :DH]), _pad(rcp_x[:, DH:]))

    invd_rcp = _deg_call(dst_ur)
    invd_usr = _deg_call(dst_ru)

    x_u = usr0
    ys_r = []
    ys_u = []
    for _ in range(N_LAYERS):
        x_r = _conv_call(x_u[0], x_u[1], src_ur, dst_ur, invd_rcp)
        ys_r.append(x_r)
        x_u = _conv_call(x_r[0], x_r[1], src_ru, dst_ru, invd_usr)
        ys_u.append(x_u)

    usr_out = _final_call(usr0[0], usr0[1], ys_u[0][0], ys_u[0][1],
                          ys_u[1][0], ys_u[1][1], ys_u[2][0], ys_u[2][1])
    rec_out = _final_call(rcp0[0], rcp0[1], ys_r[0][0], ys_r[0][1],
                          ys_r[1][0], ys_r[1][1], ys_r[2][0], ys_r[2][1])
    return (usr_out, rec_out)
